# baseline (speedup 1.0000x reference)
import jax
import jax.numpy as jnp
from jax.experimental import pallas as pl
from jax.experimental.pallas import tpu as pltpu

P = 4096
B = 6
F_IN = 960
NW = 4
BQ = 512
BP2 = 512
BK2 = 1024


def _stage1_body(e0_ref, e1_ref, e2_ref, e3_ref, refp_ref, w_emb_ref,
                 w_refp_ref, b12_ref, out_ref):
    rp = jax.lax.dot_general(w_refp_ref[:, :], refp_ref[:, :],
                             dimension_numbers=((([0]), ([1])), ((), ())),
                             preferred_element_type=jnp.float32)
    rp = rp + b12_ref[:, 0:1]
    for w, e_ref in enumerate((e0_ref, e1_ref, e2_ref, e3_ref)):
        s = jax.lax.dot_general(w_emb_ref[:, :], e_ref[:, :],
                                dimension_numbers=((([0]), ([1])), ((), ())),
                                preferred_element_type=jnp.float32)
        out_ref[:, w * BQ:(w + 1) * BQ] = s + rp[:, w * BQ:(w + 1) * BQ]


def _stage2_body(adj_ref, s36_ref, sself_ref, pm_ref, rgb_ref, acc_ref):
    l = pl.program_id(1)
    part = jnp.dot(adj_ref[:, :], s36_ref[pl.ds(l * BK2, BK2), :],
                   preferred_element_type=jnp.float32)

    @pl.when(l == 0)
    def _init():
        acc_ref[:, :] = sself_ref[:, :] + part

    @pl.when(l > 0)
    def _accum():
        acc_ref[:, :] = acc_ref[:, :] + part

    @pl.when(l == (P // BK2) - 1)
    def _emit():
        res = acc_ref[:, :]
        pm_ref[:, :] = jnp.tanh(res[:, 0:18])
        rgb_ref[:, :] = jax.nn.sigmoid(res[:, 18:36])


def kernel(embeddings, ref, adj, W_d, Wl_d, b_d, W_r, Wl_r, b_r):
    f32 = jnp.float32
    W12 = jnp.concatenate([W_d, W_r, Wl_d, Wl_r], axis=1).astype(f32)
    w_emb = W12[:F_IN, :]
    w_refp = jnp.pad(W12[F_IN:, :], ((0, 5), (0, 0)))
    refp = jnp.pad(ref[0].astype(f32), ((0, 0), (0, 5)))
    b12 = jnp.concatenate([jnp.zeros((6,), f32), b_d.astype(f32),
                           b_r.astype(f32)])
    b12t = jnp.tile(b12[:, None], (1, 128))

    emb2d = embeddings.reshape(B * P, F_IN)
    nrb = P // BQ
    grid1 = (B * P) // (NW * BQ)

    def _estream(w):
        return pl.BlockSpec((BQ, F_IN), lambda g, w=w: (g * NW + w, 0))

    s_flat = pl.pallas_call(
        _stage1_body,
        grid=(grid1,),
        in_specs=[
            _estream(0), _estream(1), _estream(2), _estream(3),
            pl.BlockSpec((NW * BQ, 8), lambda g: (g % (nrb // NW), 0)),
            pl.BlockSpec((F_IN, 12), lambda g: (0, 0)),
            pl.BlockSpec((8, 12), lambda g: (0, 0)),
            pl.BlockSpec((12, 128), lambda g: (0, 0)),
        ],
        out_specs=pl.BlockSpec((12, NW * BQ), lambda g: (0, g)),
        out_shape=jax.ShapeDtypeStruct((12, B * P), f32),
    )(emb2d, emb2d, emb2d, emb2d, refp, w_emb, w_refp, b12t)

    s3 = s_flat.reshape(12, B, P)
    sd = s3[0:3].transpose(2, 1, 0).reshape(P, 18)
    sr = s3[3:6].transpose(2, 1, 0).reshape(P, 18)
    s36 = jnp.concatenate([sd, sr], axis=1)
    ld = s3[6:9].transpose(2, 1, 0).reshape(P, 18)
    lr = s3[9:12].transpose(2, 1, 0).reshape(P, 18)
    sself = jnp.concatenate([ld, lr], axis=1)

    nb2 = P // BP2
    nl2 = P // BK2
    pm18, rgb18 = pl.pallas_call(
        _stage2_body,
        grid=(nb2, nl2),
        in_specs=[
            pl.BlockSpec((BP2, BK2), lambda j, l: (j, l)),
            pl.BlockSpec((P, 36), lambda j, l: (0, 0)),
            pl.BlockSpec((BP2, 36), lambda j, l: (j, 0)),
        ],
        out_specs=[
            pl.BlockSpec((BP2, 18), lambda j, l: (j, 0)),
            pl.BlockSpec((BP2, 18), lambda j, l: (j, 0)),
        ],
        out_shape=[
            jax.ShapeDtypeStruct((P, 18), f32),
            jax.ShapeDtypeStruct((P, 18), f32),
        ],
        scratch_shapes=[pltpu.VMEM((BP2, 36), f32)],
    )(adj.astype(f32), s36, sself)

    points_move = pm18.reshape(P, B, 3).transpose(1, 0, 2)
    rgb = rgb18.reshape(P, B, 3).transpose(1, 0, 2)
    return (points_move, rgb)


# speedup vs baseline: 1.0706x; 1.0706x over previous
import jax
import jax.numpy as jnp
from jax.experimental import pallas as pl
from jax.experimental.pallas import tpu as pltpu

P = 4096
B = 6
F_IN = 960
NW = 4
BQ = 512
BP2 = 2048
BK2 = 1024


def _stage1_body(e0_ref, e1_ref, e2_ref, e3_ref, refp_ref, w_emb_ref,
                 w_refp_ref, b12_ref, out_ref):
    rp = jax.lax.dot_general(w_refp_ref[:, :], refp_ref[:, :],
                             dimension_numbers=((([0]), ([1])), ((), ())),
                             preferred_element_type=jnp.float32)
    rp = rp + b12_ref[:, 0:1]
    for w, e_ref in enumerate((e0_ref, e1_ref, e2_ref, e3_ref)):
        s = jax.lax.dot_general(w_emb_ref[:, :], e_ref[:, :],
                                dimension_numbers=((([0]), ([1])), ((), ())),
                                preferred_element_type=jnp.float32)
        out_ref[:, w * BQ:(w + 1) * BQ] = s + rp[:, w * BQ:(w + 1) * BQ]


def _stage2_body(adj_ref, s36_ref, sself_ref, pm_ref, rgb_ref, acc_ref):
    l = pl.program_id(1)
    part = jnp.dot(adj_ref[:, :], s36_ref[pl.ds(l * BK2, BK2), :],
                   preferred_element_type=jnp.float32)

    @pl.when(l == 0)
    def _init():
        acc_ref[:, :] = sself_ref[:, :] + part

    @pl.when(l > 0)
    def _accum():
        acc_ref[:, :] = acc_ref[:, :] + part

    @pl.when(l == (P // BK2) - 1)
    def _emit():
        res = acc_ref[:, :]
        pm_ref[:, :] = jnp.tanh(res[:, 0:18])
        rgb_ref[:, :] = jax.nn.sigmoid(res[:, 18:36])


def kernel(embeddings, ref, adj, W_d, Wl_d, b_d, W_r, Wl_r, b_r):
    f32 = jnp.float32
    W12 = jnp.concatenate([W_d, W_r, Wl_d, Wl_r], axis=1).astype(f32)
    w_emb = W12[:F_IN, :]
    w_refp = jnp.pad(W12[F_IN:, :], ((0, 5), (0, 0)))
    refp = jnp.pad(ref[0].astype(f32), ((0, 0), (0, 5)))
    b12 = jnp.concatenate([jnp.zeros((6,), f32), b_d.astype(f32),
                           b_r.astype(f32)])
    b12t = jnp.tile(b12[:, None], (1, 128))

    emb2d = embeddings.reshape(B * P, F_IN)
    nrb = P // BQ
    grid1 = (B * P) // (NW * BQ)

    def _estream(w):
        return pl.BlockSpec((BQ, F_IN), lambda g, w=w: (g * NW + w, 0))

    s_flat = pl.pallas_call(
        _stage1_body,
        grid=(grid1,),
        in_specs=[
            _estream(0), _estream(1), _estream(2), _estream(3),
            pl.BlockSpec((NW * BQ, 8), lambda g: (g % (nrb // NW), 0)),
            pl.BlockSpec((F_IN, 12), lambda g: (0, 0)),
            pl.BlockSpec((8, 12), lambda g: (0, 0)),
            pl.BlockSpec((12, 128), lambda g: (0, 0)),
        ],
        out_specs=pl.BlockSpec((12, NW * BQ), lambda g: (0, g)),
        out_shape=jax.ShapeDtypeStruct((12, B * P), f32),
    )(emb2d, emb2d, emb2d, emb2d, refp, w_emb, w_refp, b12t)

    s3 = s_flat.reshape(12, B, P)
    sd = s3[0:3].transpose(2, 1, 0).reshape(P, 18)
    sr = s3[3:6].transpose(2, 1, 0).reshape(P, 18)
    s36 = jnp.concatenate([sd, sr], axis=1)
    ld = s3[6:9].transpose(2, 1, 0).reshape(P, 18)
    lr = s3[9:12].transpose(2, 1, 0).reshape(P, 18)
    sself = jnp.concatenate([ld, lr], axis=1)

    nb2 = P // BP2
    nl2 = P // BK2
    pm18, rgb18 = pl.pallas_call(
        _stage2_body,
        grid=(nb2, nl2),
        in_specs=[
            pl.BlockSpec((BP2, BK2), lambda j, l: (j, l)),
            pl.BlockSpec((P, 36), lambda j, l: (0, 0)),
            pl.BlockSpec((BP2, 36), lambda j, l: (j, 0)),
        ],
        out_specs=[
            pl.BlockSpec((BP2, 18), lambda j, l: (j, 0)),
            pl.BlockSpec((BP2, 18), lambda j, l: (j, 0)),
        ],
        out_shape=[
            jax.ShapeDtypeStruct((P, 18), f32),
            jax.ShapeDtypeStruct((P, 18), f32),
        ],
        scratch_shapes=[pltpu.VMEM((BP2, 36), f32)],
    )(adj.astype(f32), s36, sself)

    points_move = pm18.reshape(P, B, 3).transpose(1, 0, 2)
    rgb = rgb18.reshape(P, B, 3).transpose(1, 0, 2)
    return (points_move, rgb)
